# Initial kernel scaffold; baseline (speedup 1.0000x reference)
import jax
import jax.numpy as jnp
from jax.experimental import pallas as pl


def kernel(img, bbox, w_poly, b_poly, w_feat, b_feat, w_cls, b_cls, w_gcn, b_gcn):
    raise NotImplementedError("write your pallas kernel here")


# speedup vs baseline: 10.7743x; 10.7743x over previous
import jax
import jax.numpy as jnp
from jax.experimental import pallas as pl
from jax.experimental.pallas import tpu as pltpu

_BB = 8
_P = 8


def _fused_kernel(wp_ref, img_ref, idx_ref, binh_ref, wf_ref, bf_ref,
                  wc_ref, bc_ref, wg_ref, bg_ref,
                  poly_ref, cls_ref, pred_ref):
    x = img_ref[...]

    poly_ref[...] = (wp_ref[0] * x[:, 0:1, :] + wp_ref[1] * x[:, 1:2, :]
                     + wp_ref[2] * x[:, 2:3, :] + wp_ref[3])

    pooled = jnp.mean(x, axis=2)
    logits = (pooled[:, 0:1] * wc_ref[0:1, :]
              + pooled[:, 1:2] * wc_ref[1:2, :]
              + pooled[:, 2:3] * wc_ref[2:3, :] + bc_ref[...])
    cls_ref[...] = jax.nn.sigmoid(logits)

    hw = x.shape[2]
    idx = idx_ref[...]
    iota = jax.lax.broadcasted_iota(jnp.int32, (x.shape[0], idx.shape[1], hw), 2)
    sel = (iota == idx[:, :, None]).astype(jnp.float32)
    g0 = jnp.sum(sel * x[:, 0:1, :], axis=2)
    g1 = jnp.sum(sel * x[:, 1:2, :], axis=2)
    g2 = jnp.sum(sel * x[:, 2:3, :], axis=2)

    wcomb = jnp.dot(wf_ref[...], wg_ref[...],
                    preferred_element_type=jnp.float32)
    const = jnp.dot(bf_ref[...], wg_ref[...],
                    preferred_element_type=jnp.float32) + bg_ref[...]
    offs = (g0[:, :, None] * wcomb[0:1, :]
            + g1[:, :, None] * wcomb[1:2, :]
            + g2[:, :, None] * wcomb[2:3, :] + const)
    pred_ref[...] = binh_ref[...] + offs


def kernel(img, bbox, w_poly, b_poly, w_feat, b_feat, w_cls, b_cls, w_gcn, b_gcn):
    B, C, H, W = img.shape
    HW = H * W
    Hs, Ws = H // 2, W // 2
    Cf = w_feat.shape[1]
    P = _P

    img_flat = img.reshape(B, C, HW).astype(jnp.float32)

    t = jnp.linspace(0.0, 2.0 * jnp.pi, P, endpoint=False)
    ux = 0.5 + 0.45 * jnp.cos(t)
    uy = 0.5 + 0.45 * jnp.sin(t)
    bw = bbox[:, 2:3]
    bh = bbox[:, 3:4]
    hx = jnp.floor(ux[None, :] * bw)
    hy = jnp.floor(uy[None, :] * bh)
    wg_ = jnp.maximum(bw, 1e-6)
    hg_ = jnp.maximum(bh, 1e-6)
    original_hull = jnp.stack([hy, hx], axis=-1).astype(jnp.int32)
    binary_hull = jnp.stack([hy / hg_, hx / wg_], axis=-1).astype(jnp.float32)
    feature_hull = jnp.stack([jnp.floor(hy * Hs / hg_),
                              jnp.floor(hx * Ws / wg_)], axis=-1).astype(jnp.int32)

    fy = jnp.clip(feature_hull[..., 0], 0, Hs - 1)
    fx = jnp.clip(feature_hull[..., 1], 0, Ws - 1)
    px = (2 * fy * W + 2 * fx).astype(jnp.int32)

    wp = jnp.concatenate([w_poly.reshape(3),
                          b_poly.reshape(1)]).astype(jnp.float32)
    wf = w_feat.astype(jnp.float32)
    bf = b_feat.reshape(1, Cf).astype(jnp.float32)
    wc = w_cls.astype(jnp.float32)
    bc = b_cls.reshape(1, 2).astype(jnp.float32)
    wg = w_gcn.astype(jnp.float32)
    bg = b_gcn.reshape(1, 2).astype(jnp.float32)

    BB = _BB

    poly_out, cls_out, pred_out = pl.pallas_call(
        _fused_kernel,
        out_shape=(
            jax.ShapeDtypeStruct((B, 1, HW), jnp.float32),
            jax.ShapeDtypeStruct((B, 2), jnp.float32),
            jax.ShapeDtypeStruct((B, P, 2), jnp.float32),
        ),
        grid=(B // BB,),
        in_specs=[
            pl.BlockSpec(memory_space=pltpu.MemorySpace.SMEM),
            pl.BlockSpec((BB, C, HW), lambda b: (b, 0, 0)),
            pl.BlockSpec((BB, P), lambda b: (b, 0)),
            pl.BlockSpec((BB, P, 2), lambda b: (b, 0, 0)),
            pl.BlockSpec((C, Cf), lambda b: (0, 0)),
            pl.BlockSpec((1, Cf), lambda b: (0, 0)),
            pl.BlockSpec((C, 2), lambda b: (0, 0)),
            pl.BlockSpec((1, 2), lambda b: (0, 0)),
            pl.BlockSpec((Cf, 2), lambda b: (0, 0)),
            pl.BlockSpec((1, 2), lambda b: (0, 0)),
        ],
        out_specs=(
            pl.BlockSpec((BB, 1, HW), lambda b: (b, 0, 0)),
            pl.BlockSpec((BB, 2), lambda b: (b, 0)),
            pl.BlockSpec((BB, P, 2), lambda b: (b, 0, 0)),
        ),
        compiler_params=pltpu.CompilerParams(dimension_semantics=("parallel",)),
    )(wp, img_flat, px, binary_hull, wf, bf, wc, bc, wg, bg)

    poly_logits = poly_out.reshape(B, 1, H, W)
    class_prob = cls_out
    pred_polys = pred_out

    return (pred_polys, original_hull, binary_hull, feature_hull,
            poly_logits, class_prob)
